# Initial kernel scaffold; baseline (speedup 1.0000x reference)
import jax
import jax.numpy as jnp
from jax.experimental import pallas as pl


def kernel(edges, features, W1, b1, W2, b2, W3, b3, Wfc, bfc):
    raise NotImplementedError("write your pallas kernel here")


# speedup vs baseline: 17.6486x; 17.6486x over previous
import functools

import jax
import jax.numpy as jnp
from jax import lax
from jax.experimental import pallas as pl
from jax.experimental.pallas import tpu as pltpu
from jax.experimental.pallas import tpu_sc as plsc

N = 10000
R = 10240
E = 320000
DF = 128
NW = 32
CH = 128
NCH = 79
EW = NCH * CH
EP = NW * EW
STR = R // 16
BR = 256
NBLK = R // BR


def _sc_mesh():
    return plsc.VectorSubcoreMesh(core_axis_name="c", subcore_axis_name="s")


def _make_deg_kernel():
    @functools.partial(
        pl.kernel,
        mesh=_sc_mesh(),
        out_type=jax.ShapeDtypeStruct((2, R, 16), jnp.float32),
        scratch_types=[
            pltpu.VMEM((NCH, CH), jnp.int32),
            pltpu.VMEM((CH, 16), jnp.float32),
            pltpu.VMEM_SHARED((R, 16), jnp.float32),
        ],
        compiler_params=pltpu.CompilerParams(use_tc_tiling_on_sc=False),
    )
    def deg_kernel(dst_hbm, z_hbm, ones_hbm, out_hbm, dst_v, ones_v, acc):
        c = lax.axis_index("c")
        s = lax.axis_index("s")
        w = s * 2 + c
        pltpu.sync_copy(z_hbm.at[pl.ds(s * STR, STR)], acc.at[pl.ds(s * STR, STR)])
        pltpu.sync_copy(dst_hbm.at[w], dst_v)
        pltpu.sync_copy(ones_hbm, ones_v)
        plsc.subcore_barrier()

        def body(j, carry):
            pltpu.sync_copy(ones_v, acc.at[dst_v.at[j]], add=True)
            return carry

        lax.fori_loop(0, NCH, body, 0)
        plsc.subcore_barrier()
        pltpu.sync_copy(acc.at[pl.ds(s * STR, STR)],
                        out_hbm.at[c, pl.ds(s * STR, STR)])

    return deg_kernel


def _make_aggr_kernel(d):
    @functools.partial(
        pl.kernel,
        mesh=_sc_mesh(),
        out_type=jax.ShapeDtypeStruct((2, R, d), jnp.float32),
        scratch_types=[
            pltpu.VMEM((EW,), jnp.int32),
            pltpu.VMEM((NCH, CH), jnp.int32),
            pltpu.VMEM((CH, d), jnp.float32),
            pltpu.VMEM_SHARED((R, d), jnp.float32),
            pltpu.SemaphoreType.DMA,
        ],
        compiler_params=pltpu.CompilerParams(use_tc_tiling_on_sc=False),
    )
    def aggr_kernel(hs_hbm, src_hbm, dst_hbm, z_hbm, out_hbm,
                    src_v, dst_v, rows, acc, sem):
        c = lax.axis_index("c")
        s = lax.axis_index("s")
        w = s * 2 + c
        pltpu.sync_copy(z_hbm.at[pl.ds(s * STR, STR)], acc.at[pl.ds(s * STR, STR)])
        pltpu.sync_copy(src_hbm.at[pl.ds(w * EW, EW)], src_v)
        pltpu.sync_copy(dst_hbm.at[w], dst_v)
        plsc.subcore_barrier()

        def body(j, carry):
            pltpu.async_copy(hs_hbm.at[src_v.at[pl.ds(j * CH, CH)]], rows, sem).wait()
            pltpu.sync_copy(rows, acc.at[dst_v.at[j]], add=True)
            return carry

        lax.fori_loop(0, NCH, body, 0)
        plsc.subcore_barrier()
        pltpu.sync_copy(acc.at[pl.ds(s * STR, STR)],
                        out_hbm.at[c, pl.ds(s * STR, STR)])

    return aggr_kernel


def _tc_first(x_pad, w1, deg_parts):
    def body(d0_ref, d1_ref, x_ref, w_ref, hs_ref, dinv_ref):
        deg = d0_ref[:, 0:1] + d1_ref[:, 0:1] + 1.0
        dinv = lax.rsqrt(deg)
        h = jnp.dot(x_ref[...], w_ref[...], preferred_element_type=jnp.float32)
        hs_ref[...] = h * dinv
        dinv_ref[...] = dinv

    return pl.pallas_call(
        body,
        grid=(NBLK,),
        in_specs=[
            pl.BlockSpec((BR, 16), lambda i: (i, 0)),
            pl.BlockSpec((BR, 16), lambda i: (i, 0)),
            pl.BlockSpec((BR, DF), lambda i: (i, 0)),
            pl.BlockSpec((DF, 64), lambda i: (0, 0)),
        ],
        out_specs=[
            pl.BlockSpec((BR, 64), lambda i: (i, 0)),
            pl.BlockSpec((BR, 1), lambda i: (i, 0)),
        ],
        out_shape=[
            jax.ShapeDtypeStruct((R, 64), jnp.float32),
            jax.ShapeDtypeStruct((R, 1), jnp.float32),
        ],
    )(deg_parts[0], deg_parts[1], x_pad, w1)


def _tc_mid(parts, hs, dinv, b, w_next, d, d_next):
    def body(p0_ref, p1_ref, hs_ref, dinv_ref, b_ref, w_ref, f_ref, hsn_ref):
        dinv = dinv_ref[...]
        f = (p0_ref[...] + p1_ref[...] + hs_ref[...]) * dinv + b_ref[...]
        f = jnp.maximum(f, 0.0)
        f_ref[...] = f
        hsn_ref[...] = jnp.dot(f, w_ref[...],
                               preferred_element_type=jnp.float32) * dinv

    return pl.pallas_call(
        body,
        grid=(NBLK,),
        in_specs=[
            pl.BlockSpec((BR, d), lambda i: (i, 0)),
            pl.BlockSpec((BR, d), lambda i: (i, 0)),
            pl.BlockSpec((BR, d), lambda i: (i, 0)),
            pl.BlockSpec((BR, 1), lambda i: (i, 0)),
            pl.BlockSpec((1, d), lambda i: (0, 0)),
            pl.BlockSpec((d, d_next), lambda i: (0, 0)),
        ],
        out_specs=[
            pl.BlockSpec((BR, d), lambda i: (i, 0)),
            pl.BlockSpec((BR, d_next), lambda i: (i, 0)),
        ],
        out_shape=[
            jax.ShapeDtypeStruct((R, d), jnp.float32),
            jax.ShapeDtypeStruct((R, d_next), jnp.float32),
        ],
    )(parts[0], parts[1], hs, dinv, b, w_next)


def _tc_last(parts, hs3, dinv, b3, f1, f2, wfc1, wfc2, wfc3, bfc):
    def body(p0_ref, p1_ref, hs_ref, dinv_ref, b3_ref, f1_ref, f2_ref,
             w1_ref, w2_ref, w3_ref, bfc_ref, out_ref):
        f3 = (p0_ref[...] + p1_ref[...] + hs_ref[...]) * dinv_ref[...] + b3_ref[...]
        f3 = jnp.maximum(f3, 0.0)
        acc = jnp.dot(f1_ref[...], w1_ref[...], preferred_element_type=jnp.float32)
        acc += jnp.dot(f2_ref[...], w2_ref[...], preferred_element_type=jnp.float32)
        acc += jnp.dot(f3, w3_ref[...], preferred_element_type=jnp.float32)
        out_ref[...] = jnp.maximum(acc + bfc_ref[...], 0.0)

    return pl.pallas_call(
        body,
        grid=(NBLK,),
        in_specs=[
            pl.BlockSpec((BR, 16), lambda i: (i, 0)),
            pl.BlockSpec((BR, 16), lambda i: (i, 0)),
            pl.BlockSpec((BR, 16), lambda i: (i, 0)),
            pl.BlockSpec((BR, 1), lambda i: (i, 0)),
            pl.BlockSpec((1, 16), lambda i: (0, 0)),
            pl.BlockSpec((BR, 64), lambda i: (i, 0)),
            pl.BlockSpec((BR, 32), lambda i: (i, 0)),
            pl.BlockSpec((64, 16), lambda i: (0, 0)),
            pl.BlockSpec((32, 16), lambda i: (0, 0)),
            pl.BlockSpec((16, 16), lambda i: (0, 0)),
            pl.BlockSpec((1, 16), lambda i: (0, 0)),
        ],
        out_specs=pl.BlockSpec((BR, 16), lambda i: (i, 0)),
        out_shape=jax.ShapeDtypeStruct((R, 16), jnp.float32),
    )(parts[0], parts[1], hs3, dinv, b3, f1, f2, wfc1, wfc2, wfc3, bfc)


_deg_kernel = _make_deg_kernel()
_aggr64 = _make_aggr_kernel(64)
_aggr32 = _make_aggr_kernel(32)
_aggr16 = _make_aggr_kernel(16)


def kernel(edges, features, W1, b1, W2, b2, W3, b3, Wfc, bfc):
    edges = edges.astype(jnp.int32)
    pad = jnp.full((EP - E,), N, jnp.int32)
    src = jnp.concatenate([edges[0], pad])
    dst3d = jnp.concatenate([edges[1], pad]).reshape(NW, NCH, CH)

    x_pad = jnp.zeros((R, DF), jnp.float32).at[:N].set(features)
    z64 = jnp.zeros((R, 64), jnp.float32)
    z32 = jnp.zeros((R, 32), jnp.float32)
    z16 = jnp.zeros((R, 16), jnp.float32)
    ones128 = jnp.ones((CH, 16), jnp.float32)

    deg_parts = _deg_kernel(dst3d, z16, ones128)
    hs1, dinv = _tc_first(x_pad, W1, deg_parts)

    parts1 = _aggr64(hs1, src, dst3d, z64)
    f1, hs2 = _tc_mid(parts1, hs1, dinv, b1.reshape(1, 64), W2, 64, 32)

    parts2 = _aggr32(hs2, src, dst3d, z32)
    f2, hs3 = _tc_mid(parts2, hs2, dinv, b2.reshape(1, 32), W3, 32, 16)

    parts3 = _aggr16(hs3, src, dst3d, z16)
    out = _tc_last(parts3, hs3, dinv, b3.reshape(1, 16), f1, f2,
                   Wfc[:64], Wfc[64:96], Wfc[96:112], bfc.reshape(1, 16))
    return out[:N]


# speedup vs baseline: 22.2704x; 1.2619x over previous
import functools

import jax
import jax.numpy as jnp
from jax import lax
from jax.experimental import pallas as pl
from jax.experimental.pallas import tpu as pltpu
from jax.experimental.pallas import tpu_sc as plsc

N = 10000
R = 10240
E = 320000
DF = 128
NW = 32
CH = 128
NCH = 79
EW = NCH * CH
EP = NW * EW
STR = R // 16
BR = 256
NBLK = R // BR


def _sc_mesh():
    return plsc.VectorSubcoreMesh(core_axis_name="c", subcore_axis_name="s")


def _make_deg_kernel():
    @functools.partial(
        pl.kernel,
        mesh=_sc_mesh(),
        out_type=jax.ShapeDtypeStruct((2, R, 16), jnp.float32),
        scratch_types=[
            pltpu.VMEM((NCH, CH), jnp.int32),
            pltpu.VMEM((CH, 16), jnp.float32),
            pltpu.VMEM_SHARED((R, 16), jnp.float32),
        ],
        compiler_params=pltpu.CompilerParams(use_tc_tiling_on_sc=False),
    )
    def deg_kernel(dst_hbm, z_hbm, ones_hbm, out_hbm, dst_v, ones_v, acc):
        c = lax.axis_index("c")
        s = lax.axis_index("s")
        w = s * 2 + c
        pltpu.sync_copy(z_hbm.at[pl.ds(s * STR, STR)], acc.at[pl.ds(s * STR, STR)])
        pltpu.sync_copy(dst_hbm.at[w], dst_v)
        pltpu.sync_copy(ones_hbm, ones_v)
        plsc.subcore_barrier()

        def body(j, carry):
            pltpu.sync_copy(ones_v, acc.at[dst_v.at[j]], add=True)
            return carry

        lax.fori_loop(0, NCH, body, 0)
        plsc.subcore_barrier()
        pltpu.sync_copy(acc.at[pl.ds(s * STR, STR)],
                        out_hbm.at[c, pl.ds(s * STR, STR)])

    return deg_kernel


def _make_aggr_kernel(d):
    NB = 4
    LA = 3
    NQ = (NCH - LA) // NB
    TAIL = NCH - NB * NQ

    @functools.partial(
        pl.kernel,
        mesh=_sc_mesh(),
        out_type=jax.ShapeDtypeStruct((2, R, d), jnp.float32),
        scratch_types=[
            pltpu.VMEM((EW,), jnp.int32),
            pltpu.VMEM((NCH, CH), jnp.int32),
            [pltpu.VMEM((CH, d), jnp.float32)] * NB,
            [pltpu.SemaphoreType.DMA] * NB,
            pltpu.VMEM_SHARED((R, d), jnp.float32),
        ],
        compiler_params=pltpu.CompilerParams(use_tc_tiling_on_sc=False),
    )
    def aggr_kernel(hs_hbm, src_hbm, dst_hbm, z_hbm, out_hbm,
                    src_v, dst_v, rings, sems, acc):
        c = lax.axis_index("c")
        s = lax.axis_index("s")
        w = s * 2 + c
        pltpu.sync_copy(z_hbm.at[pl.ds(s * STR, STR)], acc.at[pl.ds(s * STR, STR)])
        pltpu.sync_copy(src_hbm.at[pl.ds(w * EW, EW)], src_v)
        pltpu.sync_copy(dst_hbm.at[w], dst_v)
        plsc.subcore_barrier()

        def gather(b, j):
            return pltpu.make_async_copy(
                hs_hbm.at[src_v.at[pl.ds(j * CH, CH)]], rings[b], sems[b])

        def scatter(b, j):
            pltpu.sync_copy(rings[b], acc.at[dst_v.at[j]], add=True)

        for b in range(LA):
            gather(b, b).start()

        def quad(q, carry):
            j0 = q * NB
            for b in range(NB):
                j = j0 + b
                gather(b, j).wait()
                scatter(b, j)
                gather((b + LA) % NB, j + LA).start()
            return carry

        lax.fori_loop(0, NQ, quad, 0)
        for t in range(TAIL):
            j = NB * NQ + t
            b = j % NB
            gather(b, j).wait()
            scatter(b, j)
            if j + LA < NCH:
                gather((b + LA) % NB, j + LA).start()

        plsc.subcore_barrier()
        pltpu.sync_copy(acc.at[pl.ds(s * STR, STR)],
                        out_hbm.at[c, pl.ds(s * STR, STR)])

    return aggr_kernel


def _tc_first(x_pad, w1, deg_parts):
    def body(d0_ref, d1_ref, x_ref, w_ref, hs_ref, dinv_ref):
        deg = d0_ref[:, 0:1] + d1_ref[:, 0:1] + 1.0
        dinv = lax.rsqrt(deg)
        h = jnp.dot(x_ref[...], w_ref[...], preferred_element_type=jnp.float32)
        hs_ref[...] = h * dinv
        dinv_ref[...] = dinv

    return pl.pallas_call(
        body,
        grid=(NBLK,),
        in_specs=[
            pl.BlockSpec((BR, 16), lambda i: (i, 0)),
            pl.BlockSpec((BR, 16), lambda i: (i, 0)),
            pl.BlockSpec((BR, DF), lambda i: (i, 0)),
            pl.BlockSpec((DF, 64), lambda i: (0, 0)),
        ],
        out_specs=[
            pl.BlockSpec((BR, 64), lambda i: (i, 0)),
            pl.BlockSpec((BR, 1), lambda i: (i, 0)),
        ],
        out_shape=[
            jax.ShapeDtypeStruct((R, 64), jnp.float32),
            jax.ShapeDtypeStruct((R, 1), jnp.float32),
        ],
    )(deg_parts[0], deg_parts[1], x_pad, w1)


def _tc_mid(parts, hs, dinv, b, w_next, d, d_next):
    def body(p0_ref, p1_ref, hs_ref, dinv_ref, b_ref, w_ref, f_ref, hsn_ref):
        dinv = dinv_ref[...]
        f = (p0_ref[...] + p1_ref[...] + hs_ref[...]) * dinv + b_ref[...]
        f = jnp.maximum(f, 0.0)
        f_ref[...] = f
        hsn_ref[...] = jnp.dot(f, w_ref[...],
                               preferred_element_type=jnp.float32) * dinv

    return pl.pallas_call(
        body,
        grid=(NBLK,),
        in_specs=[
            pl.BlockSpec((BR, d), lambda i: (i, 0)),
            pl.BlockSpec((BR, d), lambda i: (i, 0)),
            pl.BlockSpec((BR, d), lambda i: (i, 0)),
            pl.BlockSpec((BR, 1), lambda i: (i, 0)),
            pl.BlockSpec((1, d), lambda i: (0, 0)),
            pl.BlockSpec((d, d_next), lambda i: (0, 0)),
        ],
        out_specs=[
            pl.BlockSpec((BR, d), lambda i: (i, 0)),
            pl.BlockSpec((BR, d_next), lambda i: (i, 0)),
        ],
        out_shape=[
            jax.ShapeDtypeStruct((R, d), jnp.float32),
            jax.ShapeDtypeStruct((R, d_next), jnp.float32),
        ],
    )(parts[0], parts[1], hs, dinv, b, w_next)


def _tc_last(parts, hs3, dinv, b3, f1, f2, wfc1, wfc2, wfc3, bfc):
    def body(p0_ref, p1_ref, hs_ref, dinv_ref, b3_ref, f1_ref, f2_ref,
             w1_ref, w2_ref, w3_ref, bfc_ref, out_ref):
        f3 = (p0_ref[...] + p1_ref[...] + hs_ref[...]) * dinv_ref[...] + b3_ref[...]
        f3 = jnp.maximum(f3, 0.0)
        acc = jnp.dot(f1_ref[...], w1_ref[...], preferred_element_type=jnp.float32)
        acc += jnp.dot(f2_ref[...], w2_ref[...], preferred_element_type=jnp.float32)
        acc += jnp.dot(f3, w3_ref[...], preferred_element_type=jnp.float32)
        out_ref[...] = jnp.maximum(acc + bfc_ref[...], 0.0)

    return pl.pallas_call(
        body,
        grid=(NBLK,),
        in_specs=[
            pl.BlockSpec((BR, 16), lambda i: (i, 0)),
            pl.BlockSpec((BR, 16), lambda i: (i, 0)),
            pl.BlockSpec((BR, 16), lambda i: (i, 0)),
            pl.BlockSpec((BR, 1), lambda i: (i, 0)),
            pl.BlockSpec((1, 16), lambda i: (0, 0)),
            pl.BlockSpec((BR, 64), lambda i: (i, 0)),
            pl.BlockSpec((BR, 32), lambda i: (i, 0)),
            pl.BlockSpec((64, 16), lambda i: (0, 0)),
            pl.BlockSpec((32, 16), lambda i: (0, 0)),
            pl.BlockSpec((16, 16), lambda i: (0, 0)),
            pl.BlockSpec((1, 16), lambda i: (0, 0)),
        ],
        out_specs=pl.BlockSpec((BR, 16), lambda i: (i, 0)),
        out_shape=jax.ShapeDtypeStruct((R, 16), jnp.float32),
    )(parts[0], parts[1], hs3, dinv, b3, f1, f2, wfc1, wfc2, wfc3, bfc)


_deg_kernel = _make_deg_kernel()
_aggr64 = _make_aggr_kernel(64)
_aggr32 = _make_aggr_kernel(32)
_aggr16 = _make_aggr_kernel(16)


def kernel(edges, features, W1, b1, W2, b2, W3, b3, Wfc, bfc):
    edges = edges.astype(jnp.int32)
    pad = jnp.full((EP - E,), N, jnp.int32)
    src = jnp.concatenate([edges[0], pad])
    dst3d = jnp.concatenate([edges[1], pad]).reshape(NW, NCH, CH)

    x_pad = jnp.zeros((R, DF), jnp.float32).at[:N].set(features)
    z64 = jnp.zeros((R, 64), jnp.float32)
    z32 = jnp.zeros((R, 32), jnp.float32)
    z16 = jnp.zeros((R, 16), jnp.float32)
    ones128 = jnp.ones((CH, 16), jnp.float32)

    deg_parts = _deg_kernel(dst3d, z16, ones128)
    hs1, dinv = _tc_first(x_pad, W1, deg_parts)

    parts1 = _aggr64(hs1, src, dst3d, z64)
    f1, hs2 = _tc_mid(parts1, hs1, dinv, b1.reshape(1, 64), W2, 64, 32)

    parts2 = _aggr32(hs2, src, dst3d, z32)
    f2, hs3 = _tc_mid(parts2, hs2, dinv, b2.reshape(1, 32), W3, 32, 16)

    parts3 = _aggr16(hs3, src, dst3d, z16)
    out = _tc_last(parts3, hs3, dinv, b3.reshape(1, 16), f1, f2,
                   Wfc[:64], Wfc[64:96], Wfc[96:112], bfc.reshape(1, 16))
    return out[:N]
